# Initial kernel scaffold; baseline (speedup 1.0000x reference)
import functools

import jax
import jax.numpy as jnp
from jax import lax
from jax.experimental import pallas as pl
from jax.experimental.pallas import tpu as pltpu
from jax.experimental.pallas import tpu_sc as plsc

C = 1000
N = 1024 * 50
NC, NS = 2, 16
NW = NC * NS
B_PER_W = N // NW
CHUNK = 32
N_CHUNKS = B_PER_W // CHUNK
G16 = CHUNK // 16


def _row_lse(table):
    def body(t_ref, o_ref):
        t = t_ref[...]
        m = jnp.max(t, axis=1)
        s = jnp.sum(jnp.exp(t - m[:, None]), axis=1)
        o_ref[...] = m + jnp.log(s)

    return pl.pallas_call(
        body,
        out_shape=jax.ShapeDtypeStruct((table.shape[0],), jnp.float32),
    )(table)


_MESH = plsc.VectorSubcoreMesh(
    core_axis_name="c", subcore_axis_name="s", num_cores=NC, num_subcores=NS
)


@functools.partial(
    pl.kernel,
    out_type=[
        jax.ShapeDtypeStruct((N, C), jnp.float32),
        jax.ShapeDtypeStruct((NW, 16), jnp.float32),
    ],
    mesh=_MESH,
    scratch_types=[
        pltpu.VMEM((B_PER_W,), jnp.int32),
        pltpu.VMEM((B_PER_W,), jnp.int32),
        pltpu.VMEM((C,), jnp.float32),
        pltpu.VMEM((CHUNK, C), jnp.float32),
        pltpu.VMEM((16,), jnp.float32),
        pltpu.SemaphoreType.DMA,
    ],
)
def _sc_main(table_hbm, x_hbm, t_hbm, lse_hbm, out_hbm, part_hbm,
             idx_v, tgt_v, lse_v, buf, acc_v, gsem):
    wid = lax.axis_index("s") * NC + lax.axis_index("c")
    base = wid * B_PER_W
    pltpu.sync_copy(x_hbm.at[pl.ds(base, B_PER_W)], idx_v)
    pltpu.sync_copy(t_hbm.at[pl.ds(base, B_PER_W)], tgt_v)
    pltpu.sync_copy(lse_hbm, lse_v)
    acc_v[...] = jnp.zeros((16,), jnp.float32)

    def chunk_body(c, carry):
        start = c * CHUNK
        pltpu.async_copy(table_hbm.at[idx_v.at[pl.ds(start, CHUNK)]], buf,
                         gsem).wait()
        for g in range(G16):
            rows = lax.iota(jnp.int32, 16) + g * 16
            tv = tgt_v[pl.ds(start + g * 16, 16)]
            xv = idx_v[pl.ds(start + g * 16, 16)]
            vals = plsc.load_gather(buf, [rows, tv])
            lses = plsc.load_gather(lse_v, [xv])
            plsc.addupdate(acc_v, lses - vals)
        pltpu.sync_copy(buf, out_hbm.at[pl.ds(base + start, CHUNK)])
        return carry

    lax.fori_loop(0, N_CHUNKS, chunk_body, 0)
    pltpu.sync_copy(acc_v, part_hbm.at[wid])


def kernel(x, targets, table):
    lse = _row_lse(table)
    xf = x.reshape(-1)
    tf = targets.reshape(-1)
    logits2, partials = _sc_main(table, xf, tf, lse)
    loss = jnp.sum(partials) / jnp.float32(N)
    return (logits2, loss)


# speedup vs baseline: 1.6116x; 1.6116x over previous
import functools

import jax
import jax.numpy as jnp
from jax import lax
from jax.experimental import pallas as pl
from jax.experimental.pallas import tpu as pltpu
from jax.experimental.pallas import tpu_sc as plsc

C = 1000
N = 1024 * 50
NC, NS = 2, 16
NW = NC * NS
B_PER_W = N // NW
CHUNK = 32
N_CHUNKS = B_PER_W // CHUNK
G16 = CHUNK // 16


def _row_lse(table):
    def body(t_ref, o_ref):
        t = t_ref[...]
        m = jnp.max(t, axis=1)
        s = jnp.sum(jnp.exp(t - m[:, None]), axis=1)
        o_ref[...] = m + jnp.log(s)

    return pl.pallas_call(
        body,
        out_shape=jax.ShapeDtypeStruct((table.shape[0],), jnp.float32),
    )(table)


_MESH = plsc.VectorSubcoreMesh(
    core_axis_name="c", subcore_axis_name="s", num_cores=NC, num_subcores=NS
)


@functools.partial(
    pl.kernel,
    out_type=[
        jax.ShapeDtypeStruct((N, C), jnp.float32),
        jax.ShapeDtypeStruct((NW, 16), jnp.float32),
    ],
    mesh=_MESH,
    compiler_params=pltpu.CompilerParams(
        use_tc_tiling_on_sc=False, needs_layout_passes=False),
    scratch_types=[
        pltpu.VMEM((B_PER_W,), jnp.int32),
        pltpu.VMEM((B_PER_W,), jnp.int32),
        pltpu.VMEM((C,), jnp.float32),
        pltpu.VMEM((CHUNK, C), jnp.float32),
        pltpu.VMEM((16,), jnp.float32),
        pltpu.SemaphoreType.DMA,
    ],
)
def _sc_main(table_hbm, x_hbm, t_hbm, lse_hbm, out_hbm, part_hbm,
             idx_v, tgt_v, lse_v, buf, acc_v, gsem):
    wid = lax.axis_index("s") * NC + lax.axis_index("c")
    base = wid * B_PER_W
    pltpu.sync_copy(x_hbm.at[pl.ds(base, B_PER_W)], idx_v)
    pltpu.sync_copy(t_hbm.at[pl.ds(base, B_PER_W)], tgt_v)
    pltpu.sync_copy(lse_hbm, lse_v)
    acc_v[...] = jnp.zeros((16,), jnp.float32)

    def chunk_body(c, carry):
        start = c * CHUNK
        pltpu.async_copy(
            table_hbm.at[idx_v.at[pl.ds(start, CHUNK)]], buf, gsem).wait()
        for g in range(G16):
            rows = lax.iota(jnp.int32, 16) + g * 16
            tv = tgt_v[pl.ds(start + g * 16, 16)]
            xv = idx_v[pl.ds(start + g * 16, 16)]
            vals = plsc.load_gather(buf, [rows, tv])
            lses = plsc.load_gather(lse_v, [xv])
            acc_v[...] = acc_v[...] + (lses - vals)
        pltpu.sync_copy(buf, out_hbm.at[pl.ds(base + start, CHUNK)])
        return carry

    lax.fori_loop(0, N_CHUNKS, chunk_body, 0)
    pltpu.sync_copy(acc_v, part_hbm.at[wid])


def kernel(x, targets, table):
    lse = _row_lse(table)
    xf = x.reshape(-1)
    tf = targets.reshape(-1)
    logits2, partials = _sc_main(table, xf, tf, lse)
    loss = jnp.sum(partials) / jnp.float32(N)
    return (logits2, loss)


# speedup vs baseline: 1.7097x; 1.0609x over previous
import functools

import jax
import jax.numpy as jnp
from jax import lax
from jax.experimental import pallas as pl
from jax.experimental.pallas import tpu as pltpu
from jax.experimental.pallas import tpu_sc as plsc

C = 1000
N = 1024 * 50
NC, NS = 2, 16
NW = NC * NS
B_PER_W = N // NW
CHUNK = 32
N_CHUNKS = B_PER_W // CHUNK
G16 = CHUNK // 16


def _row_lse(table):
    def body(t_ref, o_ref):
        t = t_ref[...]
        m = jnp.max(t, axis=1)
        s = jnp.sum(jnp.exp(t - m[:, None]), axis=1)
        o_ref[...] = m + jnp.log(s)

    return pl.pallas_call(
        body,
        out_shape=jax.ShapeDtypeStruct((table.shape[0],), jnp.float32),
    )(table)


_MESH = plsc.VectorSubcoreMesh(
    core_axis_name="c", subcore_axis_name="s", num_cores=NC, num_subcores=NS
)


@functools.partial(
    pl.kernel,
    out_type=[
        jax.ShapeDtypeStruct((N, C), jnp.float32),
        jax.ShapeDtypeStruct((NW, 16), jnp.float32),
    ],
    mesh=_MESH,
    compiler_params=pltpu.CompilerParams(
        use_tc_tiling_on_sc=False, needs_layout_passes=False),
    scratch_types=[
        pltpu.VMEM((B_PER_W,), jnp.int32),
        pltpu.VMEM((B_PER_W,), jnp.int32),
        pltpu.VMEM((C,), jnp.float32),
        pltpu.VMEM((CHUNK, C), jnp.float32),
        pltpu.VMEM((CHUNK, C), jnp.float32),
        pltpu.VMEM((16,), jnp.float32),
        pltpu.SemaphoreType.DMA,
        pltpu.SemaphoreType.DMA,
        pltpu.SemaphoreType.DMA,
        pltpu.SemaphoreType.DMA,
    ],
)
def _sc_main(table_hbm, x_hbm, t_hbm, lse_hbm, out_hbm, part_hbm,
             idx_v, tgt_v, lse_v, buf0, buf1, acc_v,
             gsem0, gsem1, osem0, osem1):
    bufs = (buf0, buf1)
    gsems = (gsem0, gsem1)
    osems = (osem0, osem1)

    wid = lax.axis_index("s") * NC + lax.axis_index("c")
    base = wid * B_PER_W
    pltpu.sync_copy(x_hbm.at[pl.ds(base, B_PER_W)], idx_v)
    pltpu.sync_copy(t_hbm.at[pl.ds(base, B_PER_W)], tgt_v)
    pltpu.sync_copy(lse_hbm, lse_v)
    acc_v[...] = jnp.zeros((16,), jnp.float32)

    def start_gather(c, b):
        pltpu.async_copy(
            table_hbm.at[idx_v.at[pl.ds(c * CHUNK, CHUNK)]], bufs[b], gsems[b])

    def wait_gather(b):
        pltpu.make_async_copy(
            table_hbm.at[pl.ds(0, CHUNK)], bufs[b], gsems[b]).wait()

    def start_out(c, b):
        pltpu.async_copy(
            bufs[b], out_hbm.at[pl.ds(base + c * CHUNK, CHUNK)], osems[b])

    def wait_out(b):
        pltpu.make_async_copy(
            bufs[b], out_hbm.at[pl.ds(base, CHUNK)], osems[b]).wait()

    start_gather(0, 0)

    def outer(g, carry):
        for b in range(2):
            c = g * 2 + b
            nb = 1 - b
            @pl.when(c >= 1)
            def _():
                wait_out(nb)

            @pl.when(c + 1 < N_CHUNKS)
            def _():
                start_gather(c + 1, nb)

            wait_gather(b)
            start = c * CHUNK
            for g16 in range(G16):
                rows = lax.iota(jnp.int32, 16) + g16 * 16
                tv = tgt_v[pl.ds(start + g16 * 16, 16)]
                xv = idx_v[pl.ds(start + g16 * 16, 16)]
                vals = plsc.load_gather(bufs[b], [rows, tv])
                lses = plsc.load_gather(lse_v, [xv])
                acc_v[...] = acc_v[...] + (lses - vals)
            start_out(c, b)
        return carry

    lax.fori_loop(0, N_CHUNKS // 2, outer, 0)
    wait_out((N_CHUNKS - 1) % 2)
    pltpu.sync_copy(acc_v, part_hbm.at[wid])


def kernel(x, targets, table):
    lse = _row_lse(table)
    xf = x.reshape(-1)
    tf = targets.reshape(-1)
    logits2, partials = _sc_main(table, xf, tf, lse)
    loss = jnp.sum(partials) / jnp.float32(N)
    return (logits2, loss)
